# Initial kernel scaffold; baseline (speedup 1.0000x reference)
import jax
import jax.numpy as jnp
from jax.experimental import pallas as pl


def kernel(x, edge_index, curr_node_id, partitions, core_values, W1, b1, W2, b2, lin1_W, lin1_b, lin2_W, lin2_b):
    raise NotImplementedError("write your pallas kernel here")


# speedup vs baseline: 7.7783x; 7.7783x over previous
import functools

import jax
import jax.numpy as jnp
from jax import lax
from jax.experimental import pallas as pl
from jax.experimental.pallas import tpu as pltpu
from jax.experimental.pallas import tpu_sc as plsc

N = 10000
H = 128
NPAD = 10240
DUMP = 10200
NC = 2
NS = 16
NW = NC * NS
KE = 128
STRIPE = NPAD // NS
_EPS = 1e-5


def _mesh():
    return plsc.VectorSubcoreMesh(core_axis_name="c", subcore_axis_name="s")


NHALF = NPAD // 2
TDEG = NHALF + 128
_DSTRIPE = TDEG // NS


def _deg_sc(dstp, ch):
    @functools.partial(
        pl.kernel,
        out_type=jax.ShapeDtypeStruct((NC, TDEG, 16), jnp.float32),
        mesh=_mesh(),
        scratch_types=[
            pltpu.VMEM((ch, KE), jnp.int32),
            pltpu.VMEM((KE, 16), jnp.float32),
            pltpu.VMEM((KE, 16), jnp.float32),
            pltpu.VMEM_SHARED((TDEG, 16), jnp.float32),
        ],
    )
    def k(dstp_hbm, degp_hbm, dst_v, ones_v, zb_v, degtab):
        cid = lax.axis_index("c")
        sid = lax.axis_index("s")

        def fill(i, _):
            ones_v[i, :] = jnp.full((16,), 1.0, jnp.float32)
            zb_v[i, :] = jnp.zeros((16,), jnp.float32)
            return 0

        lax.fori_loop(0, KE, fill, 0)
        base = sid * _DSTRIPE
        off = 0
        for sz in ([KE] * (_DSTRIPE // KE) + [_DSTRIPE % KE]):
            if sz:
                pltpu.sync_copy(zb_v.at[pl.ds(0, sz)],
                                degtab.at[pl.ds(base + off, sz)])
            off += sz
        pltpu.sync_copy(dstp_hbm.at[sid], dst_v)

        offv = jnp.full((16,), cid * NHALF, jnp.int32)
        limv = jnp.full((16,), NHALF, jnp.int32)

        def adj(r, _):
            for j in range(KE // 16):
                t = dst_v[r, pl.ds(j * 16, 16)] - offv
                ok = (t >= 0) & (t < limv)
                dst_v[r, pl.ds(j * 16, 16)] = jnp.where(ok, t, limv)
            return 0

        lax.fori_loop(0, ch, adj, 0)
        plsc.subcore_barrier()

        def body(g, _):
            pltpu.sync_copy(ones_v, degtab.at[dst_v.at[g]], add=True)
            return 0

        lax.fori_loop(0, ch, body, 0)
        plsc.subcore_barrier()
        pltpu.sync_copy(degtab.at[pl.ds(base, _DSTRIPE)],
                        degp_hbm.at[cid, pl.ds(base, _DSTRIPE)])

    return k(dstp)


TAGG = NHALF + 128
_ASTRIPE = TAGG // NS


@functools.lru_cache(maxsize=None)
def _agg_sc_kernel(ch):
    @functools.partial(
        pl.kernel,
        out_type=jax.ShapeDtypeStruct((NC, TAGG, H), jnp.float32),
        mesh=_mesh(),
        scratch_types=[
            pltpu.VMEM((ch, KE), jnp.int32),
            pltpu.VMEM((ch, KE), jnp.int32),
            pltpu.VMEM((KE, H), jnp.float32),
            pltpu.VMEM((KE, H), jnp.float32),
            pltpu.VMEM_SHARED((TAGG, H), jnp.float32),
            pltpu.SemaphoreType.DMA,
            pltpu.SemaphoreType.DMA,
        ],
    )
    def k(y_hbm, srcp_hbm, dstp_hbm, accp_hbm,
          src_v, dst_v, buf0, buf1, acc, sem0, sem1):
        cid = lax.axis_index("c")
        sid = lax.axis_index("s")

        def fill(i, _):
            for j in range(H // 16):
                buf0[i, pl.ds(j * 16, 16)] = jnp.zeros((16,), jnp.float32)
            return 0

        lax.fori_loop(0, KE, fill, 0)
        base = sid * _ASTRIPE
        off = 0
        for sz in ([KE] * (_ASTRIPE // KE) + [_ASTRIPE % KE]):
            if sz:
                pltpu.sync_copy(buf0.at[pl.ds(0, sz)],
                                acc.at[pl.ds(base + off, sz)])
            off += sz
        pltpu.sync_copy(srcp_hbm.at[sid], src_v)
        pltpu.sync_copy(dstp_hbm.at[sid], dst_v)

        offv = jnp.full((16,), cid * NHALF, jnp.int32)
        limv = jnp.full((16,), NHALF, jnp.int32)

        def adj(r, _):
            for j in range(KE // 16):
                t = dst_v[r, pl.ds(j * 16, 16)] - offv
                ok = (t >= 0) & (t < limv)
                dst_v[r, pl.ds(j * 16, 16)] = jnp.where(ok, t, limv)
            return 0

        lax.fori_loop(0, ch, adj, 0)
        plsc.subcore_barrier()

        pltpu.async_copy(y_hbm.at[src_v.at[0]], buf0, sem0)

        def body(gg, _):
            g = gg * 2
            cp1 = pltpu.async_copy(y_hbm.at[src_v.at[g + 1]], buf1, sem1)
            pltpu.make_async_copy(y_hbm.at[src_v.at[g]], buf0, sem0).wait()
            pltpu.sync_copy(buf0, acc.at[dst_v.at[g]], add=True)

            @pl.when(g + 2 < ch)
            def _():
                pltpu.async_copy(y_hbm.at[src_v.at[g + 2]], buf0, sem0)

            cp1.wait()
            pltpu.sync_copy(buf1, acc.at[dst_v.at[g + 1]], add=True)
            return 0

        lax.fori_loop(0, ch // 2, body, 0)
        plsc.subcore_barrier()
        pltpu.sync_copy(acc.at[pl.ds(base, _ASTRIPE)],
                        accp_hbm.at[cid, pl.ds(base, _ASTRIPE)])

    return k


def _agg_sc(y, srcp, dstp, ch):
    return _agg_sc_kernel(ch)(y, srcp, dstp)


def _pool_sc(h2, pidx, cvp, curr16, pr):
    chunks = [KE] * (pr // KE) + ([pr % KE] if pr % KE else [])

    @functools.partial(
        pl.kernel,
        out_type=(jax.ShapeDtypeStruct((NW, 2, H), jnp.float32),
                  jax.ShapeDtypeStruct((1, H), jnp.float32)),
        mesh=_mesh(),
        scratch_types=[
            pltpu.VMEM((pr,), jnp.int32),
            pltpu.VMEM((pr,), jnp.float32),
            pltpu.VMEM((KE, H), jnp.float32),
            pltpu.VMEM((NPAD,), jnp.float32),
            pltpu.VMEM((2, H), jnp.float32),
            pltpu.VMEM((16,), jnp.int32),
            pltpu.VMEM((16, H), jnp.float32),
            pltpu.SemaphoreType.DMA,
        ],
        compiler_params=pltpu.CompilerParams(needs_layout_passes=False),
    )
    def k(h2_hbm, pidx_hbm, cvp_hbm, curr_hbm, pooled_hbm, xcurr_hbm,
          idx_v, cg_buf, rows_v, cv_v, out_v, cur_v, crow_v, sem):
        cid = lax.axis_index("c")
        sid = lax.axis_index("s")
        w = sid * NC + cid
        pltpu.sync_copy(pidx_hbm.at[w], idx_v)
        pltpu.sync_copy(cvp_hbm, cv_v)

        def cg_fill(kk, _):
            idx16 = idx_v[pl.ds(kk * 16, 16)]
            cg_buf[pl.ds(kk * 16, 16)] = plsc.load_gather(cv_v, [idx16])
            return 0

        lax.fori_loop(0, pr // 16, cg_fill, 0)

        acc = tuple(jnp.zeros((16,), jnp.float32) for _ in range(2 * (H // 16)))
        off = 0
        for size in chunks:
            pltpu.async_copy(h2_hbm.at[idx_v.at[pl.ds(off, size)]],
                             rows_v.at[pl.ds(0, size)], sem).wait()

            def row_body(r, a, off=off):
                rb = jnp.full((16,), off, jnp.int32) + r
                cg = plsc.load_gather(cg_buf, [rb])
                new_sc = []
                new_s1 = []
                for j in range(H // 16):
                    seg = rows_v[r, pl.ds(j * 16, 16)]
                    new_sc.append(a[j] + cg * seg)
                    new_s1.append(a[H // 16 + j] + seg)
                return tuple(new_sc) + tuple(new_s1)

            acc = lax.fori_loop(0, size, row_body, acc)
            off += size

        for j in range(H // 16):
            out_v[0, pl.ds(j * 16, 16)] = acc[j]
            out_v[1, pl.ds(j * 16, 16)] = acc[H // 16 + j]
        pltpu.sync_copy(out_v, pooled_hbm.at[w])

        @pl.when(w == 0)
        def _():
            pltpu.sync_copy(curr_hbm, cur_v)
            pltpu.async_copy(h2_hbm.at[cur_v], crow_v, sem).wait()
            pltpu.sync_copy(crow_v.at[pl.ds(0, 1)], xcurr_hbm)

    return k(h2, pidx, cvp, curr16)


def _xw_body(x_ref, w1_ref, degp_ref, y_ref, dinv_ref):
    d0 = degp_ref[0]
    d1 = degp_ref[1]
    deg = jnp.concatenate([d0[:NHALF, 0:1], d1[:NHALF, 0:1]], axis=0) + 1.0
    rid = lax.broadcasted_iota(jnp.int32, (NPAD, 1), 0)
    dinv = jnp.where(rid < N, lax.rsqrt(deg), 0.0)
    dinv_ref[...] = dinv
    y_ref[...] = jnp.dot(x_ref[...], w1_ref[...],
                         preferred_element_type=jnp.float32) * dinv


def _xw_tc(x_pad, w1, degp):
    return pl.pallas_call(
        _xw_body,
        out_shape=(jax.ShapeDtypeStruct((NPAD, H), jnp.float32),
                   jax.ShapeDtypeStruct((NPAD, 1), jnp.float32)),
    )(x_pad, w1, degp)


def _bn_relu(acc_ref, y_ref, dinv_ref):
    dinv = dinv_ref[...]
    acc = jnp.concatenate([acc_ref[0][:NHALF], acc_ref[1][:NHALF]], axis=0)
    out1 = dinv * (acc + y_ref[...])
    m = jnp.sum(out1, axis=0, keepdims=True) * (1.0 / N)
    rid = lax.broadcasted_iota(jnp.int32, (NPAD, H), 0)
    xc = jnp.where(rid < N, out1 - m, 0.0)
    v = jnp.sum(xc * xc, axis=0, keepdims=True) * (1.0 / N)
    return jnp.maximum(xc * lax.rsqrt(v + _EPS), 0.0)


def _bnmm_body(acc_ref, y_ref, dinv_ref, w2_ref, out_ref):
    h = _bn_relu(acc_ref, y_ref, dinv_ref)
    out_ref[...] = jnp.dot(h, w2_ref[...],
                           preferred_element_type=jnp.float32) * dinv_ref[...]


def _bnmm_tc(accp, y, dinv, w2):
    return pl.pallas_call(
        _bnmm_body,
        out_shape=jax.ShapeDtypeStruct((NPAD, H), jnp.float32),
    )(accp, y, dinv, w2)


def _bn_body(acc_ref, y_ref, dinv_ref, out_ref):
    out_ref[...] = _bn_relu(acc_ref, y_ref, dinv_ref)


def _bn_tc(accp, y, dinv):
    return pl.pallas_call(
        _bn_body,
        out_shape=jax.ShapeDtypeStruct((NPAD, H), jnp.float32),
    )(accp, y, dinv)


def _head_body(sc0, sc1, s10, s11, xc_ref, w1_ref, b1_ref, w2_ref, b2_ref,
               out_ref):
    core = sc0[...] + sc1[...]
    s1 = s10[...] + s11[...]
    nonc = s1 - core
    a1 = w1_ref[0:H, :]
    a2 = w1_ref[H:2 * H, :]
    a3 = w1_ref[2 * H:3 * H, :]
    t = jnp.dot(xc_ref[...], a1, preferred_element_type=jnp.float32)
    o = (t + jnp.dot(core, a2, preferred_element_type=jnp.float32)
         + jnp.dot(nonc, a3, preferred_element_type=jnp.float32)
         + b1_ref[...])
    o = jnp.maximum(o, 0.0)
    out_ref[...] = jnp.dot(o, w2_ref[...],
                           preferred_element_type=jnp.float32) + b2_ref[...]


def _head_tc(sc0, sc1, s10, s11, xcurr, lin1_W, lin1_b, lin2_W, lin2_b):
    return pl.pallas_call(
        _head_body,
        out_shape=jax.ShapeDtypeStruct((sc0.shape[0], lin2_W.shape[1]),
                                       jnp.float32),
    )(sc0, sc1, s10, s11, xcurr, lin1_W, lin1_b.reshape(1, lin1_b.shape[-1]),
      lin2_W, lin2_b.reshape(1, lin2_b.shape[-1]))


def kernel(x, edge_index, curr_node_id, partitions, core_values,
           W1, b1, W2, b2, lin1_W, lin1_b, lin2_W, lin2_b):
    e = edge_index.shape[1]
    p, ps = partitions.shape
    ppad = -ps % (2 * (NW // p) * 8)
    pr = p * (ps + ppad) // NW

    src = edge_index[0]
    dst = edge_index[1]
    chd = -(-e // (NS * KE))
    chd += chd % 2
    epd = NS * KE * chd
    srcp16 = jnp.concatenate(
        [src, jnp.zeros((epd - e,), jnp.int32)]).reshape(NS, chd, KE)
    dstp16 = jnp.concatenate(
        [dst, jnp.full((epd - e,), DUMP, jnp.int32)]).reshape(NS, chd, KE)
    x_pad = jnp.pad(x, ((0, NPAD - x.shape[0]), (0, 0)))
    cv_pad = jnp.pad(core_values, (0, NPAD - core_values.shape[0]))
    curr16 = jnp.full((16,), curr_node_id, jnp.int32)
    pidx = jnp.pad(partitions, ((0, 0), (0, ppad)),
                   constant_values=DUMP).reshape(NW, pr)

    degp = _deg_sc(dstp16, chd)
    y1, dinv = _xw_tc(x_pad, W1, degp)
    acc1 = _agg_sc(y1, srcp16, dstp16, chd)
    y2 = _bnmm_tc(acc1, y1, dinv, W2)
    acc2 = _agg_sc(y2, srcp16, dstp16, chd)
    h2 = _bn_tc(acc2, y2, dinv)
    pooled, xcurr = _pool_sc(h2, pidx, cv_pad, curr16, pr)
    return _head_tc(pooled[0::2, 0], pooled[1::2, 0],
                    pooled[0::2, 1], pooled[1::2, 1],
                    xcurr, lin1_W, lin1_b, lin2_W, lin2_b)


# speedup vs baseline: 9.8372x; 1.2647x over previous
import functools

import jax
import jax.numpy as jnp
from jax import lax
from jax.experimental import pallas as pl
from jax.experimental.pallas import tpu as pltpu
from jax.experimental.pallas import tpu_sc as plsc

N = 10000
H = 128
NPAD = 10240
DUMP = 10200
NC = 2
NS = 16
NW = NC * NS
KE = 128
STRIPE = NPAD // NS
_EPS = 1e-5


def _mesh():
    return plsc.VectorSubcoreMesh(core_axis_name="c", subcore_axis_name="s")


NHALF = NPAD // 2
TDEG = NHALF + 128
_DSTRIPE = TDEG // NS


def _deg_sc(dstp, ch):
    @functools.partial(
        pl.kernel,
        out_type=jax.ShapeDtypeStruct((NC, TDEG, 16), jnp.float32),
        mesh=_mesh(),
        scratch_types=[
            pltpu.VMEM((ch, KE), jnp.int32),
            pltpu.VMEM((KE, 16), jnp.float32),
            pltpu.VMEM((KE, 16), jnp.float32),
            pltpu.VMEM_SHARED((TDEG, 16), jnp.float32),
        ],
    )
    def k(dstp_hbm, degp_hbm, dst_v, ones_v, zb_v, degtab):
        cid = lax.axis_index("c")
        sid = lax.axis_index("s")

        def fill(i, _):
            ones_v[i, :] = jnp.full((16,), 1.0, jnp.float32)
            zb_v[i, :] = jnp.zeros((16,), jnp.float32)
            return 0

        lax.fori_loop(0, KE, fill, 0)
        base = sid * _DSTRIPE
        off = 0
        for sz in ([KE] * (_DSTRIPE // KE) + [_DSTRIPE % KE]):
            if sz:
                pltpu.sync_copy(zb_v.at[pl.ds(0, sz)],
                                degtab.at[pl.ds(base + off, sz)])
            off += sz
        pltpu.sync_copy(dstp_hbm.at[sid], dst_v)

        offv = jnp.full((16,), cid * NHALF, jnp.int32)
        limv = jnp.full((16,), NHALF, jnp.int32)
        dumpv = (jnp.full((16,), NHALF, jnp.int32)
                 + (sid % 8) * 16 + lax.iota(jnp.int32, 16))

        def adj(r, _):
            for j in range(KE // 16):
                t = dst_v[r, pl.ds(j * 16, 16)] - offv
                ok = (t >= 0) & (t < limv)
                dst_v[r, pl.ds(j * 16, 16)] = jnp.where(ok, t, dumpv)
            return 0

        lax.fori_loop(0, ch, adj, 0)
        plsc.subcore_barrier()

        def body(g, _):
            pltpu.sync_copy(ones_v, degtab.at[dst_v.at[g]], add=True)
            return 0

        lax.fori_loop(0, ch, body, 0)
        plsc.subcore_barrier()
        pltpu.sync_copy(degtab.at[pl.ds(base, _DSTRIPE)],
                        degp_hbm.at[cid, pl.ds(base, _DSTRIPE)])

    return k(dstp)


TAGG = NHALF + 128
_ASTRIPE = TAGG // NS


@functools.lru_cache(maxsize=None)
def _agg_sc_kernel(ch):
    @functools.partial(
        pl.kernel,
        out_type=jax.ShapeDtypeStruct((NC, TAGG, H), jnp.float32),
        mesh=_mesh(),
        scratch_types=[
            pltpu.VMEM((ch, KE), jnp.int32),
            pltpu.VMEM((ch, KE), jnp.int32),
            pltpu.VMEM((KE, H), jnp.float32),
            pltpu.VMEM((KE, H), jnp.float32),
            pltpu.VMEM_SHARED((TAGG, H), jnp.float32),
            pltpu.SemaphoreType.DMA,
            pltpu.SemaphoreType.DMA,
        ],
    )
    def k(y_hbm, srcp_hbm, dstp_hbm, accp_hbm,
          src_v, dst_v, buf0, buf1, acc, sem0, sem1):
        cid = lax.axis_index("c")
        sid = lax.axis_index("s")

        def fill(i, _):
            for j in range(H // 16):
                buf0[i, pl.ds(j * 16, 16)] = jnp.zeros((16,), jnp.float32)
            return 0

        lax.fori_loop(0, KE, fill, 0)
        base = sid * _ASTRIPE
        off = 0
        for sz in ([KE] * (_ASTRIPE // KE) + [_ASTRIPE % KE]):
            if sz:
                pltpu.sync_copy(buf0.at[pl.ds(0, sz)],
                                acc.at[pl.ds(base + off, sz)])
            off += sz
        pltpu.sync_copy(srcp_hbm.at[sid], src_v)
        pltpu.sync_copy(dstp_hbm.at[sid], dst_v)

        offv = jnp.full((16,), cid * NHALF, jnp.int32)
        limv = jnp.full((16,), NHALF, jnp.int32)
        dumpv = (jnp.full((16,), NHALF, jnp.int32)
                 + (sid % 8) * 16 + lax.iota(jnp.int32, 16))

        def adj(r, _):
            for j in range(KE // 16):
                t = dst_v[r, pl.ds(j * 16, 16)] - offv
                ok = (t >= 0) & (t < limv)
                dst_v[r, pl.ds(j * 16, 16)] = jnp.where(ok, t, dumpv)
            return 0

        lax.fori_loop(0, ch, adj, 0)
        plsc.subcore_barrier()

        pltpu.async_copy(y_hbm.at[src_v.at[0]], buf0, sem0)

        def body(gg, _):
            g = gg * 2
            cp1 = pltpu.async_copy(y_hbm.at[src_v.at[g + 1]], buf1, sem1)
            pltpu.make_async_copy(y_hbm.at[src_v.at[g]], buf0, sem0).wait()
            pltpu.sync_copy(buf0, acc.at[dst_v.at[g]], add=True)

            @pl.when(g + 2 < ch)
            def _():
                pltpu.async_copy(y_hbm.at[src_v.at[g + 2]], buf0, sem0)

            cp1.wait()
            pltpu.sync_copy(buf1, acc.at[dst_v.at[g + 1]], add=True)
            return 0

        lax.fori_loop(0, ch // 2, body, 0)
        plsc.subcore_barrier()
        pltpu.sync_copy(acc.at[pl.ds(base, _ASTRIPE)],
                        accp_hbm.at[cid, pl.ds(base, _ASTRIPE)])

    return k


def _agg_sc(y, srcp, dstp, ch):
    return _agg_sc_kernel(ch)(y, srcp, dstp)


def _pool_sc(h2, pidx, cvp, curr16, pr):
    chunks = [KE] * (pr // KE) + ([pr % KE] if pr % KE else [])

    @functools.partial(
        pl.kernel,
        out_type=(jax.ShapeDtypeStruct((NW, 2, H), jnp.float32),
                  jax.ShapeDtypeStruct((1, H), jnp.float32)),
        mesh=_mesh(),
        scratch_types=[
            pltpu.VMEM((pr,), jnp.int32),
            pltpu.VMEM((pr,), jnp.float32),
            pltpu.VMEM((KE, H), jnp.float32),
            pltpu.VMEM((NPAD,), jnp.float32),
            pltpu.VMEM((2, H), jnp.float32),
            pltpu.VMEM((16,), jnp.int32),
            pltpu.VMEM((16, H), jnp.float32),
            pltpu.SemaphoreType.DMA,
        ],
        compiler_params=pltpu.CompilerParams(needs_layout_passes=False),
    )
    def k(h2_hbm, pidx_hbm, cvp_hbm, curr_hbm, pooled_hbm, xcurr_hbm,
          idx_v, cg_buf, rows_v, cv_v, out_v, cur_v, crow_v, sem):
        cid = lax.axis_index("c")
        sid = lax.axis_index("s")
        w = sid * NC + cid
        pltpu.sync_copy(pidx_hbm.at[w], idx_v)
        pltpu.sync_copy(cvp_hbm, cv_v)

        def cg_fill(kk, _):
            idx16 = idx_v[pl.ds(kk * 16, 16)]
            cg_buf[pl.ds(kk * 16, 16)] = plsc.load_gather(cv_v, [idx16])
            return 0

        lax.fori_loop(0, pr // 16, cg_fill, 0)

        acc = tuple(jnp.zeros((16,), jnp.float32) for _ in range(2 * (H // 16)))
        off = 0
        for size in chunks:
            pltpu.async_copy(h2_hbm.at[idx_v.at[pl.ds(off, size)]],
                             rows_v.at[pl.ds(0, size)], sem).wait()

            def row_body(r, a, off=off):
                rb = jnp.full((16,), off, jnp.int32) + r
                cg = plsc.load_gather(cg_buf, [rb])
                new_sc = []
                new_s1 = []
                for j in range(H // 16):
                    seg = rows_v[r, pl.ds(j * 16, 16)]
                    new_sc.append(a[j] + cg * seg)
                    new_s1.append(a[H // 16 + j] + seg)
                return tuple(new_sc) + tuple(new_s1)

            acc = lax.fori_loop(0, size, row_body, acc)
            off += size

        for j in range(H // 16):
            out_v[0, pl.ds(j * 16, 16)] = acc[j]
            out_v[1, pl.ds(j * 16, 16)] = acc[H // 16 + j]
        pltpu.sync_copy(out_v, pooled_hbm.at[w])

        @pl.when(w == 0)
        def _():
            pltpu.sync_copy(curr_hbm, cur_v)
            pltpu.async_copy(h2_hbm.at[cur_v], crow_v, sem).wait()
            pltpu.sync_copy(crow_v.at[pl.ds(0, 1)], xcurr_hbm)

    return k(h2, pidx, cvp, curr16)


def _xw_body(x_ref, w1_ref, degp_ref, y_ref, dinv_ref):
    d0 = degp_ref[0]
    d1 = degp_ref[1]
    deg = jnp.concatenate([d0[:NHALF, 0:1], d1[:NHALF, 0:1]], axis=0) + 1.0
    rid = lax.broadcasted_iota(jnp.int32, (NPAD, 1), 0)
    dinv = jnp.where(rid < N, lax.rsqrt(deg), 0.0)
    dinv_ref[...] = dinv
    y_ref[...] = jnp.dot(x_ref[...], w1_ref[...],
                         preferred_element_type=jnp.float32) * dinv


def _xw_tc(x_pad, w1, degp):
    return pl.pallas_call(
        _xw_body,
        out_shape=(jax.ShapeDtypeStruct((NPAD, H), jnp.float32),
                   jax.ShapeDtypeStruct((NPAD, 1), jnp.float32)),
    )(x_pad, w1, degp)


def _bn_relu(acc_ref, y_ref, dinv_ref):
    dinv = dinv_ref[...]
    acc = jnp.concatenate([acc_ref[0][:NHALF], acc_ref[1][:NHALF]], axis=0)
    out1 = dinv * (acc + y_ref[...])
    m = jnp.sum(out1, axis=0, keepdims=True) * (1.0 / N)
    rid = lax.broadcasted_iota(jnp.int32, (NPAD, H), 0)
    xc = jnp.where(rid < N, out1 - m, 0.0)
    v = jnp.sum(xc * xc, axis=0, keepdims=True) * (1.0 / N)
    return jnp.maximum(xc * lax.rsqrt(v + _EPS), 0.0)


def _bnmm_body(acc_ref, y_ref, dinv_ref, w2_ref, out_ref):
    h = _bn_relu(acc_ref, y_ref, dinv_ref)
    out_ref[...] = jnp.dot(h, w2_ref[...],
                           preferred_element_type=jnp.float32) * dinv_ref[...]


def _bnmm_tc(accp, y, dinv, w2):
    return pl.pallas_call(
        _bnmm_body,
        out_shape=jax.ShapeDtypeStruct((NPAD, H), jnp.float32),
    )(accp, y, dinv, w2)


def _bn_body(acc_ref, y_ref, dinv_ref, out_ref):
    out_ref[...] = _bn_relu(acc_ref, y_ref, dinv_ref)


def _bn_tc(accp, y, dinv):
    return pl.pallas_call(
        _bn_body,
        out_shape=jax.ShapeDtypeStruct((NPAD, H), jnp.float32),
    )(accp, y, dinv)


def _head_body(sc0, sc1, s10, s11, xc_ref, w1_ref, b1_ref, w2_ref, b2_ref,
               out_ref):
    core = sc0[...] + sc1[...]
    s1 = s10[...] + s11[...]
    nonc = s1 - core
    a1 = w1_ref[0:H, :]
    a2 = w1_ref[H:2 * H, :]
    a3 = w1_ref[2 * H:3 * H, :]
    t = jnp.dot(xc_ref[...], a1, preferred_element_type=jnp.float32)
    o = (t + jnp.dot(core, a2, preferred_element_type=jnp.float32)
         + jnp.dot(nonc, a3, preferred_element_type=jnp.float32)
         + b1_ref[...])
    o = jnp.maximum(o, 0.0)
    out_ref[...] = jnp.dot(o, w2_ref[...],
                           preferred_element_type=jnp.float32) + b2_ref[...]


def _head_tc(sc0, sc1, s10, s11, xcurr, lin1_W, lin1_b, lin2_W, lin2_b):
    return pl.pallas_call(
        _head_body,
        out_shape=jax.ShapeDtypeStruct((sc0.shape[0], lin2_W.shape[1]),
                                       jnp.float32),
    )(sc0, sc1, s10, s11, xcurr, lin1_W, lin1_b.reshape(1, lin1_b.shape[-1]),
      lin2_W, lin2_b.reshape(1, lin2_b.shape[-1]))


def kernel(x, edge_index, curr_node_id, partitions, core_values,
           W1, b1, W2, b2, lin1_W, lin1_b, lin2_W, lin2_b):
    e = edge_index.shape[1]
    p, ps = partitions.shape
    ppad = -ps % (2 * (NW // p) * 8)
    pr = p * (ps + ppad) // NW

    src = edge_index[0]
    dst = edge_index[1]
    chd = -(-e // (NS * KE))
    chd += chd % 2
    epd = NS * KE * chd
    srcp16 = jnp.concatenate(
        [src, jnp.zeros((epd - e,), jnp.int32)]).reshape(NS, chd, KE)
    pad_dst = N + 16 + (jnp.arange(epd - e, dtype=jnp.int32) % 128)
    dstp16 = jnp.concatenate(
        [dst, pad_dst]).reshape(NS, chd, KE)
    x_pad = jnp.pad(x, ((0, NPAD - x.shape[0]), (0, 0)))
    cv_pad = jnp.pad(core_values, (0, NPAD - core_values.shape[0]))
    curr16 = jnp.full((16,), curr_node_id, jnp.int32)
    pidx = jnp.pad(partitions, ((0, 0), (0, ppad)),
                   constant_values=DUMP).reshape(NW, pr)

    degp = _deg_sc(dstp16, chd)
    y1, dinv = _xw_tc(x_pad, W1, degp)
    acc1 = _agg_sc(y1, srcp16, dstp16, chd)
    y2 = _bnmm_tc(acc1, y1, dinv, W2)
    acc2 = _agg_sc(y2, srcp16, dstp16, chd)
    h2 = _bn_tc(acc2, y2, dinv)
    pooled, xcurr = _pool_sc(h2, pidx, cv_pad, curr16, pr)
    return _head_tc(pooled[0::2, 0], pooled[1::2, 0],
                    pooled[0::2, 1], pooled[1::2, 1],
                    xcurr, lin1_W, lin1_b, lin2_W, lin2_b)
